# baseline (speedup 1.0000x reference)
import functools

import jax
import jax.numpy as jnp
from jax import lax
from jax.experimental import pallas as pl
from jax.experimental.pallas import tpu as pltpu
from jax.experimental.pallas import tpu_sc as plsc

_D = 128
_ROWS = 1440
_NC = 2
_NS = 16
_NW = _NC * _NS


def _fuse_body(tt_ref, ht_ref, mt_ref, w_ref, b_ref, ti_ref, out_ref, idx_ref):
    idx_ref[...] = ti_ref[...].reshape(idx_ref.shape)
    wt = w_ref[0:128, :]
    wh = w_ref[128:192, :]
    wm = w_ref[192:256, :]
    ttw = jnp.dot(tt_ref[...], wt, preferred_element_type=jnp.float32)
    hc = jnp.dot(ht_ref[...], wh, preferred_element_type=jnp.float32)
    mc = jnp.dot(mt_ref[...], wm, preferred_element_type=jnp.float32)
    row = lax.broadcasted_iota(jnp.int32, (_ROWS, 1), 0)
    eh = (row // 60 == lax.broadcasted_iota(jnp.int32, (_ROWS, 24), 1)).astype(jnp.float32)
    em = (row % 60 == lax.broadcasted_iota(jnp.int32, (_ROWS, 60), 1)).astype(jnp.float32)
    out_ref[...] = (ttw
                    + jnp.dot(eh, hc, preferred_element_type=jnp.float32)
                    + jnp.dot(em, mc, preferred_element_type=jnp.float32)
                    + b_ref[...])


@functools.lru_cache(maxsize=None)
def _make_fuse(bt, seq):
    return pl.pallas_call(
        _fuse_body,
        out_shape=(
            jax.ShapeDtypeStruct((_ROWS, _D), jnp.float32),
            jax.ShapeDtypeStruct((_NW, bt // _NW, seq), jnp.int32),
        ),
    )


@functools.lru_cache(maxsize=None)
def _make_gather(bt, seq):
    rpw = bt // _NW
    mesh = plsc.VectorSubcoreMesh(
        core_axis_name="c", subcore_axis_name="s",
        num_cores=_NC, num_subcores=_NS)

    @functools.partial(
        pl.kernel,
        mesh=mesh,
        out_type=jax.ShapeDtypeStruct((bt, seq, _D), jnp.float32),
        scratch_types=[
            pltpu.VMEM((rpw, seq), jnp.int32),
            pltpu.VMEM((4, seq, _D), jnp.float32),
            pltpu.VMEM((4, seq, _D), jnp.float32),
            pltpu.SemaphoreType.DMA,
            pltpu.SemaphoreType.DMA,
            pltpu.SemaphoreType.DMA,
            pltpu.SemaphoreType.DMA,
        ],
    )
    def gather(table_hbm, idx_hbm, out_hbm, idx_v, buf0, buf1, g0, g1, s0, s1):
        wid = lax.axis_index("s") * _NC + lax.axis_index("c")
        base = wid * rpw
        pltpu.sync_copy(idx_hbm.at[wid], idx_v)

        def drain(buf, sem):
            pltpu.make_async_copy(buf, out_hbm.at[pl.ds(base, 4)], sem).wait()

        def octet(g, carry):
            r0 = 8 * g

            @pl.when(g > 0)
            def _():
                drain(buf0, s0)
                drain(buf1, s1)

            cps0 = [pltpu.async_copy(table_hbm.at[idx_v.at[r0 + k]], buf0.at[k], g0)
                    for k in range(4)]
            cps1 = [pltpu.async_copy(table_hbm.at[idx_v.at[r0 + 4 + k]], buf1.at[k], g1)
                    for k in range(4)]
            for cp in cps0:
                cp.wait()
            pltpu.async_copy(buf0, out_hbm.at[pl.ds(base + r0, 4)], s0)
            for cp in cps1:
                cp.wait()
            pltpu.async_copy(buf1, out_hbm.at[pl.ds(base + r0 + 4, 4)], s1)
            return carry

        lax.fori_loop(0, rpw // 8, octet, 0)
        drain(buf0, s0)
        drain(buf1, s1)

    return gather


def kernel(time_indices, time_table, hour_table, minute_table, W, b):
    bt, seq = time_indices.shape
    fused, idx = _make_fuse(bt, seq)(
        time_table, hour_table, minute_table, W,
        b.reshape(1, _D).astype(jnp.float32), time_indices.astype(jnp.int32))
    return _make_gather(bt, seq)(fused, idx)


# speedup vs baseline: 1.0160x; 1.0160x over previous
import functools

import jax
import jax.numpy as jnp
from jax import lax
from jax.experimental import pallas as pl
from jax.experimental.pallas import tpu as pltpu
from jax.experimental.pallas import tpu_sc as plsc

_D = 128
_ROWS = 1440
_NC = 2
_NS = 16
_NW = _NC * _NS


def _fuse_body(tt_ref, ht_ref, mt_ref, w_ref, b_ref, out_ref):
    wt = w_ref[0:128, :]
    wh = w_ref[128:192, :]
    wm = w_ref[192:256, :]
    ttw = jnp.dot(tt_ref[...], wt, preferred_element_type=jnp.float32)
    hc = jnp.dot(ht_ref[...], wh, preferred_element_type=jnp.float32)
    mc = jnp.dot(mt_ref[...], wm, preferred_element_type=jnp.float32)
    row = lax.broadcasted_iota(jnp.int32, (_ROWS, 1), 0)
    eh = (row // 60 == lax.broadcasted_iota(jnp.int32, (_ROWS, 24), 1)).astype(jnp.float32)
    em = (row % 60 == lax.broadcasted_iota(jnp.int32, (_ROWS, 60), 1)).astype(jnp.float32)
    out_ref[...] = (ttw
                    + jnp.dot(eh, hc, preferred_element_type=jnp.float32)
                    + jnp.dot(em, mc, preferred_element_type=jnp.float32)
                    + b_ref[...])


_fuse = pl.pallas_call(
    _fuse_body,
    out_shape=jax.ShapeDtypeStruct((_ROWS, _D), jnp.float32),
)


@functools.lru_cache(maxsize=None)
def _make_gather(bt, seq):
    rpw = bt // _NW
    mesh = plsc.VectorSubcoreMesh(
        core_axis_name="c", subcore_axis_name="s",
        num_cores=_NC, num_subcores=_NS)

    @functools.partial(
        pl.kernel,
        mesh=mesh,
        out_type=jax.ShapeDtypeStruct((bt, seq, _D), jnp.float32),
        scratch_types=[
            pltpu.VMEM((rpw, seq), jnp.int32),
            pltpu.VMEM((4, seq, _D), jnp.float32),
            pltpu.VMEM((4, seq, _D), jnp.float32),
            pltpu.SemaphoreType.DMA,
            pltpu.SemaphoreType.DMA,
            pltpu.SemaphoreType.DMA,
            pltpu.SemaphoreType.DMA,
        ],
    )
    def gather(table_hbm, idx_hbm, out_hbm, idx_v, buf0, buf1, g0, g1, s0, s1):
        wid = lax.axis_index("s") * _NC + lax.axis_index("c")
        base = wid * rpw
        pltpu.sync_copy(idx_hbm.at[wid], idx_v)

        def drain(buf, sem):
            pltpu.make_async_copy(buf, out_hbm.at[pl.ds(base, 4)], sem).wait()

        def octet(g, carry):
            r0 = 8 * g

            @pl.when(g > 0)
            def _():
                drain(buf0, s0)
                drain(buf1, s1)

            cps0 = [pltpu.async_copy(table_hbm.at[idx_v.at[r0 + k]], buf0.at[k], g0)
                    for k in range(4)]
            cps1 = [pltpu.async_copy(table_hbm.at[idx_v.at[r0 + 4 + k]], buf1.at[k], g1)
                    for k in range(4)]
            for cp in cps0:
                cp.wait()
            pltpu.async_copy(buf0, out_hbm.at[pl.ds(base + r0, 4)], s0)
            for cp in cps1:
                cp.wait()
            pltpu.async_copy(buf1, out_hbm.at[pl.ds(base + r0 + 4, 4)], s1)
            return carry

        lax.fori_loop(0, rpw // 8, octet, 0)
        drain(buf0, s0)
        drain(buf1, s1)

    return gather


def kernel(time_indices, time_table, hour_table, minute_table, W, b):
    bt, seq = time_indices.shape
    fused = _fuse(time_table, hour_table, minute_table, W,
                  b.reshape(1, _D).astype(jnp.float32))
    idx = time_indices.astype(jnp.int32).reshape(_NW, bt // _NW, seq)
    return _make_gather(bt, seq)(fused, idx)
